# baseline (speedup 1.0000x reference)
import functools

import jax
import jax.numpy as jnp
from jax import lax
from jax.experimental import pallas as pl
from jax.experimental.pallas import tpu as pltpu
from jax.experimental.pallas import tpu_sc as plsc

BATCH = 16
N_HEADS = 16
HEAD_DIM = 128
ROWS = BATCH * N_HEADS


@functools.lru_cache(maxsize=None)
def _build(seq_len: int):
    info = plsc.get_sparse_core_info()
    num_cores, num_subcores = info.num_cores, info.num_subcores
    num_workers = num_cores * num_subcores
    rows_per_w = ROWS // num_workers
    mesh = plsc.VectorSubcoreMesh(core_axis_name="c", subcore_axis_name="s")

    @functools.partial(
        pl.kernel,
        mesh=mesh,
        out_type=(
            jax.ShapeDtypeStruct((ROWS, HEAD_DIM), jnp.float32),
            jax.ShapeDtypeStruct((ROWS, HEAD_DIM), jnp.float32),
        ),
        scratch_types=[
            pltpu.VMEM((16,), jnp.int32),
            pltpu.VMEM((rows_per_w, HEAD_DIM), jnp.float32),
            pltpu.VMEM((rows_per_w, HEAD_DIM), jnp.float32),
        ],
    )
    def sc_kernel(new_k, new_v, k_cache, v_cache, pos_hbm,
                  out_k, out_v, pos_v, buf_k, buf_v):
        wid = lax.axis_index("s") * num_cores + lax.axis_index("c")
        r0 = wid * rows_per_w
        rows = pl.ds(r0, rows_per_w)

        pltpu.sync_copy(pos_hbm, pos_v)
        pos = pos_v[...][0]

        @pl.when(pos <= 0)
        def _():
            pltpu.sync_copy(new_k.at[rows], buf_k)
            pltpu.sync_copy(new_v.at[rows], buf_v)

        @pl.when(pos > 0)
        def _():
            pltpu.sync_copy(k_cache.at[rows, pl.ds(0, HEAD_DIM)], buf_k)
            pltpu.sync_copy(v_cache.at[rows, pl.ds(0, HEAD_DIM)], buf_v)

        pltpu.sync_copy(buf_k, out_k.at[rows])
        pltpu.sync_copy(buf_v, out_v.at[rows])

    return sc_kernel


def kernel(new_keys, new_values, k_cache, v_cache, cache_pos):
    b, h, t, d = new_keys.shape
    nk = new_keys.reshape(ROWS, HEAD_DIM)
    nv = new_values.reshape(ROWS, HEAD_DIM)
    kc = k_cache.reshape(ROWS, -1)
    vc = v_cache.reshape(ROWS, -1)
    pos = jnp.full((16,), cache_pos, dtype=jnp.int32)
    ok, ov = _build(k_cache.shape[2])(nk, nv, kc, vc, pos)
    return (ok.reshape(b, h, t, d), ov.reshape(b, h, t, d))


# speedup vs baseline: 19.3650x; 19.3650x over previous
import functools

import jax
import jax.numpy as jnp
from jax import lax
from jax.experimental import pallas as pl
from jax.experimental.pallas import tpu as pltpu
from jax.experimental.pallas import tpu_sc as plsc

BATCH = 16
N_HEADS = 16
HEAD_DIM = 128
ROWS = BATCH * N_HEADS


@functools.lru_cache(maxsize=None)
def _build(seq_len: int):
    info = plsc.get_sparse_core_info()
    num_cores, num_subcores = info.num_cores, info.num_subcores
    num_workers = num_cores * num_subcores
    rows_per_w = ROWS // num_workers
    mesh = plsc.VectorSubcoreMesh(core_axis_name="c", subcore_axis_name="s")

    @functools.partial(
        pl.kernel,
        mesh=mesh,
        out_type=(
            jax.ShapeDtypeStruct((ROWS, 1, HEAD_DIM), jnp.float32),
            jax.ShapeDtypeStruct((ROWS, 1, HEAD_DIM), jnp.float32),
        ),
        scratch_types=[
            pltpu.VMEM((16,), jnp.int32),
            pltpu.VMEM((rows_per_w, 1, HEAD_DIM), jnp.float32),
            pltpu.VMEM((rows_per_w, 1, HEAD_DIM), jnp.float32),
        ],
    )
    def sc_kernel(new_k, new_v, k_cache, v_cache, pos_hbm,
                  out_k, out_v, pos_v, buf_k, buf_v):
        wid = lax.axis_index("s") * num_cores + lax.axis_index("c")
        r0 = wid * rows_per_w
        rows = pl.ds(r0, rows_per_w)

        pltpu.sync_copy(pos_hbm, pos_v)
        pos = pos_v[...][0]

        @pl.when(pos <= 0)
        def _():
            pltpu.sync_copy(new_k.at[rows], buf_k)
            pltpu.sync_copy(new_v.at[rows], buf_v)

        @pl.when(pos > 0)
        def _():
            pltpu.sync_copy(k_cache.at[rows, pl.ds(0, 1)], buf_k)
            pltpu.sync_copy(v_cache.at[rows, pl.ds(0, 1)], buf_v)

        pltpu.sync_copy(buf_k, out_k.at[rows])
        pltpu.sync_copy(buf_v, out_v.at[rows])

    return sc_kernel


def kernel(new_keys, new_values, k_cache, v_cache, cache_pos):
    b, h, t, d = new_keys.shape
    seq_len = k_cache.shape[2]
    nk = new_keys.reshape(ROWS, 1, HEAD_DIM)
    nv = new_values.reshape(ROWS, 1, HEAD_DIM)
    kc = k_cache.reshape(ROWS, seq_len, HEAD_DIM)
    vc = v_cache.reshape(ROWS, seq_len, HEAD_DIM)
    pos = jnp.full((16,), cache_pos, dtype=jnp.int32)
    ok, ov = _build(seq_len)(nk, nv, kc, vc, pos)
    return (ok.reshape(b, h, t, d), ov.reshape(b, h, t, d))
